# Initial kernel scaffold; baseline (speedup 1.0000x reference)
import jax
import jax.numpy as jnp
from jax.experimental import pallas as pl


def kernel(geometry, euler, trans, cam, tris, vert_tris):
    raise NotImplementedError("write your pallas kernel here")


# speedup vs baseline: 6.6254x; 6.6254x over previous
import functools

import jax
import jax.numpy as jnp
from jax import lax
from jax.experimental import pallas as pl
from jax.experimental.pallas import tpu as pltpu
from jax.experimental.pallas import tpu_sc as plsc

B, N, T = 4, 100000, 200000
NW = 32
NPAD = 102400
TPAD = 204800
NB_W = TPAD // NW
NC_W = NPAD // NW
CH_B = 640
CH_C = 640
GB = 128

_GDN = lax.GatherDimensionNumbers(
    offset_dims=(), collapsed_slice_dims=(0,), start_index_map=(0,))


def _perm_vectors():
    io = lax.iota(jnp.int32, 16)
    grp = io & jnp.int32(~3)
    q = io & jnp.int32(3)
    p1 = grp + jnp.where(q == 3, jnp.int32(3), lax.rem(q + 1, jnp.int32(3)))
    p2 = grp + jnp.where(q == 3, jnp.int32(3), lax.rem(q + 2, jnp.int32(3)))
    return p1, p2


def _perm(x, idx):
    return lax.gather(x, idx[:, None], _GDN, (1,),
                      mode=lax.GatherScatterMode.PROMISE_IN_BOUNDS)


def _rsqrt_fast(x):
    i = lax.bitcast_convert_type(x, jnp.int32)
    i = jnp.int32(0x5F3759DF) - lax.shift_right_logical(i, 1)
    y = lax.bitcast_convert_type(i, jnp.float32)
    y = y * (jnp.float32(1.5) - jnp.float32(0.5) * x * y * y)
    y = y * (jnp.float32(1.5) - jnp.float32(0.5) * x * y * y)
    return y


_BLK = 2048


def _dense_body(params_ref, g_ref, pt_ref, rt_ref, ut_ref):
    g = g_ref[...].astype(jnp.float32)
    for b in range(B):
        def p(k):
            return params_ref[b, k]
        gx = g[3 * b:3 * b + 1, :]
        gy = g[3 * b + 1:3 * b + 2, :]
        gz = g[3 * b + 2:3 * b + 3, :]
        xr = p(0) * gx + p(1) * gy + p(2) * gz + p(9)
        yr = p(3) * gx + p(4) * gy + p(5) * gz + p(10)
        zr = p(6) * gx + p(7) * gy + p(8) * gz + p(11)
        px = xr / zr * p(12) + p(14)
        py = yr / zr * p(13) + p(15)
        inv = lax.rsqrt(xr * xr + yr * yr + zr * zr)
        rt_ref[pl.ds(3 * b, 1), :] = xr
        rt_ref[pl.ds(3 * b + 1, 1), :] = yr
        rt_ref[pl.ds(3 * b + 2, 1), :] = zr
        pt_ref[pl.ds(3 * b, 1), :] = px
        pt_ref[pl.ds(3 * b + 1, 1), :] = py
        pt_ref[pl.ds(3 * b + 2, 1), :] = zr
        ut_ref[pl.ds(3 * b, 1), :] = xr * inv
        ut_ref[pl.ds(3 * b + 1, 1), :] = yr * inv
        ut_ref[pl.ds(3 * b + 2, 1), :] = zr * inv


def _dense_stage(params, gt):
    grid = (NPAD // _BLK,)
    out = jax.ShapeDtypeStruct((3 * B, NPAD), jnp.float32)
    return pl.pallas_call(
        _dense_body,
        grid=grid,
        in_specs=[
            pl.BlockSpec((B, 16), lambda i: (0, 0),
                         memory_space=pltpu.SMEM),
            pl.BlockSpec((3 * B, _BLK), lambda i: (0, i)),
        ],
        out_specs=[pl.BlockSpec((3 * B, _BLK), lambda i: (0, i))] * 3,
        out_shape=[out, out, out],
    )(params, gt)


_MESH = plsc.VectorSubcoreMesh(core_axis_name="c", subcore_axis_name="s")


def _tri_body(vtab, tris3d, ntab, i0_v, i1_v, i2_v, v1_v, v2_v, v3_v,
              nout_v, sem):
    wid = lax.axis_index("s") * 2 + lax.axis_index("c")
    rpw = NB_W // GB
    gpc = CH_B // GB
    p1, p2 = _perm_vectors()
    for k, idx_v in enumerate((i0_v, i1_v, i2_v)):
        pltpu.sync_copy(tris3d.at[k, pl.ds(wid * rpw, rpw)], idx_v)

    def chunk(c, _):
        copies = []
        for k, (idx_v, dst) in enumerate(
                ((i0_v, v1_v), (i1_v, v2_v), (i2_v, v3_v))):
            for j in range(gpc):
                copies.append(pltpu.make_async_copy(
                    vtab.at[idx_v.at[c * gpc + j]],
                    dst.at[pl.ds(j * GB, GB)], sem))
        for cp in copies:
            cp.start()
        for cp in copies:
            cp.wait()

        def row(r, _):
            v1 = v1_v[r]
            e1 = v2_v[r] - v1
            e2 = v3_v[r] - v1
            n = (_perm(e1, p1) * _perm(e2, p2)
                 - _perm(e1, p2) * _perm(e2, p1))
            nn = n * n
            s = nn + _perm(nn, p1) + _perm(nn, p2)
            r_ = _rsqrt_fast(jnp.maximum(s, jnp.float32(1e-30)))
            nout_v[r] = n * r_
            return 0
        lax.fori_loop(0, CH_B, row, 0, unroll=4)
        pltpu.sync_copy(nout_v,
                        ntab.at[pl.ds(wid * NB_W + c * CH_B, CH_B)])
        return 0
    lax.fori_loop(0, NB_W // CH_B, chunk, 0)


@functools.partial(
    pl.kernel, mesh=_MESH,
    compiler_params=pltpu.CompilerParams(use_tc_tiling_on_sc=False, needs_layout_passes=False),
    out_type=jax.ShapeDtypeStruct((TPAD, 16), jnp.float32),
    scratch_types=[
        pltpu.VMEM((NB_W // GB, GB), jnp.int32),
        pltpu.VMEM((NB_W // GB, GB), jnp.int32),
        pltpu.VMEM((NB_W // GB, GB), jnp.int32),
        pltpu.VMEM((CH_B, 16), jnp.float32),
        pltpu.VMEM((CH_B, 16), jnp.float32),
        pltpu.VMEM((CH_B, 16), jnp.float32),
        pltpu.VMEM((CH_B, 16), jnp.float32),
        pltpu.SemaphoreType.DMA,
    ],
)
def _tri_kernel(vtab, tris3d, ntab, *rest):
    _tri_body(vtab, tris3d, ntab, *rest)


def _vis_body(ntab, gtab, vt3d, vis, vt_v, m_v, g_v, s_v, o_v, sem):
    wid = lax.axis_index("s") * 2 + lax.axis_index("c")
    rpw = NC_W // GB
    gpc = CH_C // GB
    p1, p2 = _perm_vectors()
    iota = lax.iota(jnp.int32, 16)
    pltpu.sync_copy(vt3d.at[pl.ds(wid * rpw, rpw)], vt_v)

    def chunk(c, _):
        base = wid * NC_W + c * CH_C
        copies = [pltpu.make_async_copy(
            ntab.at[vt_v.at[c * gpc + j]],
            m_v.at[pl.ds(j * GB, GB)], sem) for j in range(gpc)]
        for cp in copies:
            cp.start()
        pltpu.sync_copy(gtab.at[pl.ds(base, CH_C)], g_v)
        for cp in copies:
            cp.wait()

        def row(r, _):
            d = m_v[r] * g_v[r]
            s = d + _perm(d, p1) + _perm(d, p2)
            v = jnp.float32(0.0) - s
            s_v[r] = jnp.where(v < jnp.float32(0.01), jnp.float32(-1.0), v)
            return 0
        lax.fori_loop(0, CH_C, row, 0, unroll=4)

        def ext(j, _):
            rows = j * 16 + iota
            for b in range(B):
                col = plsc.load_gather(
                    s_v, [rows, jnp.full((16,), 4 * b, jnp.int32)])
                o_v[b, pl.ds(j * 16, 16)] = col
            return 0
        lax.fori_loop(0, CH_C // 16, ext, 0, unroll=2)
        for b in range(B):
            pltpu.sync_copy(o_v.at[b], vis.at[b, pl.ds(base, CH_C)])
        return 0
    lax.fori_loop(0, NC_W // CH_C, chunk, 0)


@functools.partial(
    pl.kernel, mesh=_MESH,
    compiler_params=pltpu.CompilerParams(use_tc_tiling_on_sc=False, needs_layout_passes=False),
    out_type=jax.ShapeDtypeStruct((B, NPAD), jnp.float32),
    scratch_types=[
        pltpu.VMEM((NC_W // GB, GB), jnp.int32),
        pltpu.VMEM((CH_C, 16), jnp.float32),
        pltpu.VMEM((CH_C, 16), jnp.float32),
        pltpu.VMEM((CH_C, 16), jnp.float32),
        pltpu.VMEM((B, CH_C), jnp.float32),
        pltpu.SemaphoreType.DMA,
    ],
)
def _vis_kernel(ntab, gtab, vt3d, vis, *rest):
    _vis_body(ntab, gtab, vt3d, vis, *rest)


def _rot_mats(euler):
    x, y, z = euler[:, 0], euler[:, 1], euler[:, 2]
    cx, sx = jnp.cos(x), jnp.sin(x)
    cy, sy = jnp.cos(y), jnp.sin(y)
    cz, sz = jnp.cos(z), jnp.sin(z)
    zo = jnp.zeros_like(x)
    on = jnp.ones_like(x)
    rx = jnp.stack([on, zo, zo, zo, cx, -sx, zo, sx, cx], -1).reshape(-1, 3, 3)
    ry = jnp.stack([cy, zo, sy, zo, on, zo, -sy, zo, cy], -1).reshape(-1, 3, 3)
    rz = jnp.stack([cz, -sz, zo, sz, cz, zo, zo, zo, on], -1).reshape(-1, 3, 3)
    return rz @ ry @ rx


def kernel(geometry, euler, trans, cam, tris, vert_tris):
    r = _rot_mats(euler)
    ru = lax.bitcast_convert_type(r.reshape(B, 9), jnp.int32)
    ru = (ru + 0x7FFF + (lax.shift_right_logical(ru, 16) & 1)) & jnp.int32(
        0xFFFF0000 - (1 << 32))
    r9 = lax.bitcast_convert_type(ru, jnp.float32)
    params = jnp.concatenate(
        [r9, trans,
         cam[:, 0:1], cam[:, 1:2], cam[:, 2:3], cam[:, 3:4]], axis=1)

    gt = jnp.swapaxes(geometry, 1, 2).reshape(3 * B, N)
    gt = jnp.pad(gt, ((0, 0), (0, NPAD - N))).astype(jnp.bfloat16)
    pt, rt, ut = _dense_stage(params, gt)

    def pack16(a):
        a = a.reshape(B, 3, NPAD).transpose(2, 0, 1)
        return jnp.pad(a, ((0, 0), (0, 0), (0, 1))).reshape(NPAD, 16)
    vtab = pack16(rt)
    gtab = pack16(ut)

    tris_p = jnp.pad(tris, ((0, TPAD - T), (0, 0)))
    tris3d = jnp.swapaxes(tris_p, 0, 1).reshape(3, TPAD // GB, GB)
    ntab = _tri_kernel(vtab, tris3d)

    vt3d = jnp.pad(vert_tris, (0, NPAD - N)).reshape(NPAD // GB, GB)
    vis = _vis_kernel(ntab, gtab, vt3d)

    proj_geo = pt[:, :N].reshape(B, 3, N).transpose(0, 2, 1)
    rot_tri_normal = ntab[:T].reshape(T, B, 4)[:, :, :3].transpose(1, 0, 2)
    is_visible = vis[:, :N]
    return (proj_geo, rot_tri_normal, is_visible)
